# baseline (speedup 1.0000x reference)
import functools

import jax
import jax.numpy as jnp
from jax import lax
from jax.experimental import pallas as pl
from jax.experimental.pallas import tpu as pltpu
from jax.experimental.pallas import tpu_sc as plsc

_VOCAB = 1000
_DIM = 64
_BATCH = 4096
_HIST = 200

_NC = 2
_NS = 16
_NW = _NC * _NS

_B_PER_W = _BATCH // _NW
_BG = _B_PER_W // 16
_HPAIRS = _HIST // 2

_mesh = plsc.VectorSubcoreMesh(core_axis_name="c", subcore_axis_name="s")


@functools.partial(
    pl.kernel,
    out_type=jax.ShapeDtypeStruct((_HIST, _DIM, _BATCH), jnp.float32),
    mesh=_mesh,
    scratch_types=[
        pltpu.VMEM((_VOCAB * 72,), jnp.float32),
        pltpu.VMEM((_HIST, _B_PER_W), jnp.int32),
        pltpu.VMEM((2, _DIM, _B_PER_W), jnp.float32),
        pltpu.VMEM((2, _DIM, _B_PER_W), jnp.float32),
        pltpu.SemaphoreType.DMA,
        pltpu.SemaphoreType.DMA,
    ],
    compiler_params=pltpu.CompilerParams(use_tc_tiling_on_sc=True,
                                         needs_layout_passes=False),
)
def _emb_lookup(idx_hbm, table_hbm, out_hbm,
                table_v, ids_v, rows0, rows1, sem_w0, sem_w1):
    wid = lax.axis_index("s") * _NC + lax.axis_index("c")
    b0 = wid * _B_PER_W

    pltpu.sync_copy(table_hbm, table_v)
    pltpu.sync_copy(idx_hbm.at[:, pl.ds(b0, _B_PER_W)], ids_v)

    def fill_chunk(hp, rows_v):
        for h2 in range(2):
            h = hp * 2 + h2
            for bg in range(0, _BG, 4):
                addrs = [ids_v[h, pl.ds((bg + j) * 16, 16)] * 72
                         for j in range(4)]

                @plsc.parallel_loop(0, _DIM, unroll=2)
                def _(c):
                    for j in range(4):
                        rows_v[h2, c, pl.ds((bg + j) * 16, 16)] = (
                            plsc.load_gather(table_v, [addrs[j] + c]))

    def fire_writeout(hp, rows_v, sem):
        pltpu.async_copy(
            rows_v, out_hbm.at[pl.ds(hp * 2, 2), :, pl.ds(b0, _B_PER_W)], sem)

    def wait_writeout(hp, rows_v, sem):
        pltpu.make_async_copy(
            rows_v, out_hbm.at[pl.ds(hp * 2, 2), :, pl.ds(b0, _B_PER_W)],
            sem).wait()

    def body(t, carry):
        a = 2 * t
        b = a + 1

        @pl.when(t > 0)
        def _():
            wait_writeout(a - 2, rows0, sem_w0)

        fill_chunk(a, rows0)
        fire_writeout(a, rows0, sem_w0)

        @pl.when(t > 0)
        def _():
            wait_writeout(b - 2, rows1, sem_w1)

        fill_chunk(b, rows1)
        fire_writeout(b, rows1, sem_w1)
        return carry

    lax.fori_loop(0, _HPAIRS // 2, body, 0)
    wait_writeout(_HPAIRS - 2, rows0, sem_w0)
    wait_writeout(_HPAIRS - 1, rows1, sem_w1)


def kernel(vocab_ids, table):
    ids_t = vocab_ids.astype(jnp.int32).T
    table_p = jnp.pad(table, ((0, 0), (0, 8))).reshape(_VOCAB * 72)
    out_t = _emb_lookup(ids_t, table_p)
    return jnp.transpose(out_t, (2, 0, 1))


# speedup vs baseline: 1.1120x; 1.1120x over previous
import functools

import jax
import jax.numpy as jnp
from jax import lax
from jax.experimental import pallas as pl
from jax.experimental.pallas import tpu as pltpu
from jax.experimental.pallas import tpu_sc as plsc

_VOCAB = 1000
_DIM = 64
_BATCH = 4096
_HIST = 200

_NC = 2
_NS = 16
_NW = _NC * _NS

_B_PER_W = _BATCH // _NW
_BG = _B_PER_W // 16
_HPAIRS = _HIST // 2

_mesh = plsc.VectorSubcoreMesh(core_axis_name="c", subcore_axis_name="s")


@functools.partial(
    pl.kernel,
    out_type=jax.ShapeDtypeStruct((_HIST, _DIM, _BATCH), jnp.float32),
    mesh=_mesh,
    scratch_types=[
        pltpu.VMEM((_VOCAB * 72,), jnp.float32),
        pltpu.VMEM((_HIST, _B_PER_W), jnp.int32),
        pltpu.VMEM((2, _DIM, _B_PER_W), jnp.float32),
        pltpu.VMEM((2, _DIM, _B_PER_W), jnp.float32),
        pltpu.SemaphoreType.DMA,
        pltpu.SemaphoreType.DMA,
    ],
    compiler_params=pltpu.CompilerParams(use_tc_tiling_on_sc=True,
                                         needs_layout_passes=False),
)
def _emb_lookup(idx_hbm, table_hbm, out_hbm,
                table_v, ids_v, rows0, rows1, sem_w0, sem_w1):
    wid = lax.axis_index("s") * _NC + lax.axis_index("c")
    b0 = wid * _B_PER_W

    pltpu.sync_copy(table_hbm, table_v)
    pltpu.sync_copy(idx_hbm.at[:, pl.ds(b0, _B_PER_W)], ids_v)

    def fill_chunk(hp, rows_v):
        for h2 in range(2):
            h = hp * 2 + h2
            for bg in range(0, _BG, 2):
                addr_a = ids_v[h, pl.ds(bg * 16, 16)] * 72
                addr_b = ids_v[h, pl.ds(bg * 16 + 16, 16)] * 72

                @plsc.parallel_loop(0, _DIM, unroll=4)
                def _(c):
                    rows_v[h2, c, pl.ds(bg * 16, 16)] = plsc.load_gather(
                        table_v, [addr_a + c])
                    rows_v[h2, c, pl.ds(bg * 16 + 16, 16)] = plsc.load_gather(
                        table_v, [addr_b + c])

    def fire_writeout(hp, rows_v, sem):
        pltpu.async_copy(
            rows_v, out_hbm.at[pl.ds(hp * 2, 2), :, pl.ds(b0, _B_PER_W)], sem)

    def wait_writeout(hp, rows_v, sem):
        pltpu.make_async_copy(
            rows_v, out_hbm.at[pl.ds(hp * 2, 2), :, pl.ds(b0, _B_PER_W)],
            sem).wait()

    def body(t, carry):
        a = 2 * t
        b = a + 1

        @pl.when(t > 0)
        def _():
            wait_writeout(a - 2, rows0, sem_w0)

        fill_chunk(a, rows0)
        fire_writeout(a, rows0, sem_w0)

        @pl.when(t > 0)
        def _():
            wait_writeout(b - 2, rows1, sem_w1)

        fill_chunk(b, rows1)
        fire_writeout(b, rows1, sem_w1)
        return carry

    lax.fori_loop(0, _HPAIRS // 2, body, 0)
    wait_writeout(_HPAIRS - 2, rows0, sem_w0)
    wait_writeout(_HPAIRS - 1, rows1, sem_w1)


def kernel(vocab_ids, table):
    ids_t = vocab_ids.astype(jnp.int32).T
    table_p = jnp.pad(table, ((0, 0), (0, 8))).reshape(_VOCAB * 72)
    out_t = _emb_lookup(ids_t, table_p)
    return jnp.transpose(out_t, (2, 0, 1))
